# baseline (speedup 1.0000x reference)
import functools

import jax
import jax.numpy as jnp
from jax import lax
from jax.experimental import pallas as pl
from jax.experimental.pallas import tpu as pltpu
from jax.experimental.pallas import tpu_sc as plsc


def _sc_gather(table, ids):
    V, D = table.shape
    B = ids.shape[0]
    info = plsc.get_sparse_core_info()
    nc, ns = info.num_cores, info.num_subcores
    nw = nc * ns
    b_per_w = B // nw

    mesh = plsc.VectorSubcoreMesh(core_axis_name="c", subcore_axis_name="s")

    @functools.partial(
        pl.kernel,
        mesh=mesh,
        out_type=jax.ShapeDtypeStruct((B, D), jnp.float32),
        scratch_types=[
            pltpu.VMEM((b_per_w,), jnp.int32),
            pltpu.VMEM((b_per_w, D), jnp.float32),
            pltpu.SemaphoreType.DMA,
        ],
    )
    def k(table_hbm, idx_hbm, out_hbm, idx_v, rows_v, sem):
        wid = lax.axis_index("s") * nc + lax.axis_index("c")
        base = wid * b_per_w
        pltpu.sync_copy(idx_hbm.at[pl.ds(base, b_per_w)], idx_v)
        pltpu.async_copy(table_hbm.at[idx_v], rows_v, sem).wait()
        pltpu.sync_copy(rows_v, out_hbm.at[pl.ds(base, b_per_w)])

    return k(table, ids)


def _table_prep(Wm, DO):
    DE, V = Wm.shape

    def body(w_ref, out_ref):
        eshift = (jax.lax.broadcasted_iota(jnp.int32, (DE, DO), 0) + (DO - DE)
                  == jax.lax.broadcasted_iota(jnp.int32, (DE, DO), 1)
                  ).astype(jnp.float32)
        out_ref[...] = jax.lax.dot_general(
            w_ref[...], eshift, (((0,), (0,)), ((), ())),
            preferred_element_type=jnp.float32)

    return pl.pallas_call(
        body,
        out_shape=jax.ShapeDtypeStruct((V, DO), jnp.float32),
    )(Wm)


def _assemble(lc_t, gc, b):
    W, DL, B = lc_t.shape
    DO = gc.shape[1]
    NBI = 16
    NBO = 16

    def body(lc_hbm, gc_ref, b_ref, out_hbm, lcbuf, outbuf, gcbuf, insem, outsem):
        DB = b_ref.shape[1]
        eshift = (jax.lax.broadcasted_iota(jnp.int32, (DB, DO), 0) + DL
                  == jax.lax.broadcasted_iota(jnp.int32, (DB, DO), 1)
                  ).astype(jnp.float32)
        brow = jax.lax.dot_general(
            b_ref[...], eshift, (((1,), (0,)), ((), ())),
            preferred_element_type=jnp.float32)
        gcbuf[...] = gc_ref[...] + brow
        for s in range(NBI):
            pltpu.make_async_copy(lc_hbm.at[s], lcbuf.at[s], insem.at[s]).start()

        def step(w, carry):
            slot = jax.lax.rem(w, NBI)
            oslot = jax.lax.rem(w, NBO)
            pltpu.make_async_copy(lc_hbm.at[w], lcbuf.at[slot], insem.at[slot]).wait()

            @pl.when(w >= NBO)
            def _():
                pltpu.make_async_copy(
                    outbuf.at[oslot], out_hbm.at[w - NBO], outsem.at[oslot]).wait()

            x = lcbuf[slot]
            eye = (jax.lax.broadcasted_iota(jnp.int32, (DL, DO), 0)
                   == jax.lax.broadcasted_iota(jnp.int32, (DL, DO), 1)
                   ).astype(x.dtype)
            xt = jax.lax.dot_general(
                x, eye, (((0,), (0,)), ((), ())),
                preferred_element_type=jnp.float32)
            outbuf[oslot] = xt + gcbuf[...]
            pltpu.make_async_copy(
                outbuf.at[oslot], out_hbm.at[w], outsem.at[oslot]).start()

            @pl.when(w + NBI < W)
            def _():
                nslot = jax.lax.rem(w + NBI, NBI)
                pltpu.make_async_copy(
                    lc_hbm.at[w + NBI], lcbuf.at[nslot], insem.at[nslot]).start()

            return carry

        jax.lax.fori_loop(0, W, step, 0)
        for k in range(W - NBO, W):
            s = k % NBO
            pltpu.make_async_copy(outbuf.at[s], out_hbm.at[k], outsem.at[s]).wait()

    return pl.pallas_call(
        body,
        in_specs=[
            pl.BlockSpec(memory_space=pl.ANY),
            pl.BlockSpec((B, DO), lambda: (0, 0)),
            pl.BlockSpec((1, b.shape[1]), lambda: (0, 0)),
        ],
        out_specs=pl.BlockSpec(memory_space=pl.ANY),
        out_shape=jax.ShapeDtypeStruct((W, B, DO), lc_t.dtype),
        scratch_shapes=[
            pltpu.VMEM((NBI, DL, B), jnp.float32),
            pltpu.VMEM((NBO, B, DO), jnp.float32),
            pltpu.VMEM((B, DO), jnp.float32),
            pltpu.SemaphoreType.DMA((NBI,)),
            pltpu.SemaphoreType.DMA((NBO,)),
        ],
    )(lc_t, gc, b)


def kernel(lc, ids, W, b):
    DL = lc.shape[2]
    DO = DL + W.shape[0]
    table = _table_prep(W, DO)
    gc = _sc_gather(table, ids.astype(jnp.int32))
    lc_t = jnp.transpose(lc, (1, 2, 0))
    out_t = _assemble(lc_t, gc, b.reshape(1, -1))
    return jnp.transpose(out_t, (1, 0, 2))
